# baseline (speedup 1.0000x reference)
import numpy as np
import jax
import jax.numpy as jnp
from jax import lax
from jax.experimental import pallas as pl

_B = 64
_V = 100000
_SUB = 8
_W = 1024
_NCH = _V // _W
_TAIL = _V - _NCH * _W

_KS = (np.uint32(0), np.uint32(42), np.uint32(0x1BD11BDA ^ 42))
_ROTS = ((13, 15, 26, 6), (17, 29, 16, 24))
_TINY = np.float32(np.finfo(np.float32).tiny)
_BIG_IDX = np.int32(0x7FFFFFFF)


def _rotl(x, d):
    return lax.shift_left(x, np.uint32(d)) | lax.shift_right_logical(
        x, np.uint32(32 - d)
    )


def _threefry_xor(cnt):
    x1 = cnt + _KS[1]
    x0 = x1
    x1 = x0 ^ _rotl(x1, 13)
    for r in (15, 26, 6):
        x0 = x0 + x1
        x1 = x0 ^ _rotl(x1, r)
    x0 = x0 + _KS[1]
    x1 = x1 + np.uint32(_KS[2] + np.uint32(1))
    for blk in range(1, 5):
        for r in _ROTS[blk % 2]:
            x0 = x0 + x1
            x1 = x0 ^ _rotl(x1, r)
        x0 = x0 + _KS[(blk + 1) % 3]
        x1 = x1 + np.uint32(_KS[(blk + 2) % 3] + np.uint32(blk + 1))
    return x0 ^ x1


def _race_d(xs, cnt_i32, ntinv2):
    bits = _threefry_xor(cnt_i32.astype(jnp.uint32))
    fl = (
        lax.bitcast_convert_type(
            lax.shift_right_logical(bits, np.uint32(9)) | np.uint32(0x3F800000),
            jnp.float32,
        )
        - np.float32(1.0)
    )
    return jnp.log2(fl) * jnp.exp2(xs * ntinv2)


def _rowgroup_body(x_ref, tinv_ref, o_ref):
    rg = pl.program_id(0)
    ntinv2 = tinv_ref[...]
    row0 = rg * np.int32(_SUB)
    s_iota = lax.broadcasted_iota(jnp.int32, (_SUB, _W), 0)
    l_iota = lax.broadcasted_iota(jnp.int32, (_SUB, _W), 1)
    base = (row0 + s_iota) * np.int32(_V)
    cnt0 = base + l_iota

    vmax = jnp.full((_SUB, _W), np.float32(-np.inf), jnp.float32)
    vcnt = cnt0
    for h in range(_NCH):
        col = np.int32(h * _W)
        xs = x_ref[:, h * _W : (h + 1) * _W]
        cnt = cnt0 + col
        d = _race_d(xs, cnt, ntinv2)
        upd = d > vmax
        vmax = jnp.where(upd, d, vmax)
        vcnt = jnp.where(upd, cnt, vcnt)

    m1 = jnp.max(vmax, axis=1, keepdims=True)
    sel1 = jnp.min(
        jnp.where(vmax == m1, vcnt, _BIG_IDX), axis=1, keepdims=True
    )

    xs_t = x_ref[:, _NCH * _W : _V]
    s_t = lax.broadcasted_iota(jnp.int32, (_SUB, _TAIL), 0)
    l_t = lax.broadcasted_iota(jnp.int32, (_SUB, _TAIL), 1)
    cnt_t = (row0 + s_t) * np.int32(_V) + l_t + np.int32(_NCH * _W)
    d_t = _race_d(xs_t, cnt_t, ntinv2)
    m2 = jnp.max(d_t, axis=1, keepdims=True)
    sel2 = jnp.min(
        jnp.where(d_t == m2, cnt_t, _BIG_IDX), axis=1, keepdims=True
    )

    sel = jnp.where(
        m2 > m1, sel2, jnp.where(m1 > m2, sel1, jnp.minimum(sel1, sel2))
    )
    col = sel - (row0 + lax.broadcasted_iota(jnp.int32, (_SUB, 1), 0)) * np.int32(_V)
    o_ref[...] = jnp.broadcast_to(col, (_SUB, 128))


def kernel(logits, temperatures):
    tinv = (np.float32(-np.log2(np.e)) / temperatures).reshape(_B, 1)
    out = pl.pallas_call(
        _rowgroup_body,
        grid=(_B // _SUB,),
        in_specs=[
            pl.BlockSpec((_SUB, _V), lambda g: (g, 0)),
            pl.BlockSpec((_SUB, 1), lambda g: (g, 0)),
        ],
        out_specs=pl.BlockSpec((_SUB, 128), lambda g: (g, 0)),
        out_shape=jax.ShapeDtypeStruct((_B, 128), jnp.int32),
    )(logits, tinv)
    return out[:, 0]


# speedup vs baseline: 1.0115x; 1.0115x over previous
import numpy as np
import jax
import jax.numpy as jnp
from jax import lax
from jax.experimental import pallas as pl

_B = 64
_V = 100000
_SUB = 8
_W = 1024
_NCH = _V // _W
_TAIL = _V - _NCH * _W

_KS = (np.uint32(0), np.uint32(42), np.uint32(0x1BD11BDA ^ 42))
_ROTS = ((13, 15, 26, 6), (17, 29, 16, 24))
_TINY = np.float32(np.finfo(np.float32).tiny)
_BIG_IDX = np.int32(0x7FFFFFFF)


def _rotl(x, d):
    return lax.shift_left(x, np.uint32(d)) | lax.shift_right_logical(
        x, np.uint32(32 - d)
    )


def _threefry_xor(cnt):
    x1 = cnt + _KS[1]
    x0 = x1
    x1 = x0 ^ _rotl(x1, 13)
    for r in (15, 26, 6):
        x0 = x0 + x1
        x1 = x0 ^ _rotl(x1, r)
    x0 = x0 + _KS[1]
    x1 = x1 + np.uint32(_KS[2] + np.uint32(1))
    for blk in range(1, 5):
        for r in _ROTS[blk % 2]:
            x0 = x0 + x1
            x1 = x0 ^ _rotl(x1, r)
        x0 = x0 + _KS[(blk + 1) % 3]
        x1 = x1 + np.uint32(_KS[(blk + 2) % 3] + np.uint32(blk + 1))
    return x0 ^ x1


def _race_d(xs, cnt_i32, ntinv2):
    bits = _threefry_xor(cnt_i32.astype(jnp.uint32))
    fl = (
        lax.bitcast_convert_type(
            lax.shift_right_logical(bits, np.uint32(9)) | np.uint32(0x3F800000),
            jnp.float32,
        )
        - np.float32(1.0)
    )
    return jnp.log2(fl) * jnp.exp2(xs * ntinv2)


def _rowgroup_body(x_ref, tinv_ref, o_ref):
    rg = pl.program_id(0)
    ntinv2 = tinv_ref[...]
    row0 = rg * np.int32(_SUB)
    s_iota = lax.broadcasted_iota(jnp.int32, (_SUB, _W), 0)
    l_iota = lax.broadcasted_iota(jnp.int32, (_SUB, _W), 1)
    base = (row0 + s_iota) * np.int32(_V)
    cnt0 = base + l_iota

    def body(j, carry):
        vmax, vcnt = carry
        for h in range(8):
            col = j * np.int32(8 * _W) + np.int32(h * _W)
            xs = x_ref[:, pl.ds(col, _W)]
            cnt = cnt0 + col
            d = _race_d(xs, cnt, ntinv2)
            upd = d > vmax
            vmax = jnp.where(upd, d, vmax)
            vcnt = jnp.where(upd, cnt, vcnt)
        return (vmax, vcnt)

    vmax0 = jnp.full((_SUB, _W), np.float32(-np.inf), jnp.float32)
    vmax, vcnt = lax.fori_loop(0, (_NCH - 1) // 8, body, (vmax0, cnt0))

    col96 = np.int32((_NCH - 1) * _W)
    xs96 = x_ref[:, (_NCH - 1) * _W : _NCH * _W]
    cnt96 = cnt0 + col96
    d96 = _race_d(xs96, cnt96, ntinv2)
    upd = d96 > vmax
    vmax = jnp.where(upd, d96, vmax)
    vcnt = jnp.where(upd, cnt96, vcnt)

    m1 = jnp.max(vmax, axis=1, keepdims=True)
    sel1 = jnp.min(
        jnp.where(vmax == m1, vcnt, _BIG_IDX), axis=1, keepdims=True
    )

    xs_t = x_ref[:, _NCH * _W : _V]
    s_t = lax.broadcasted_iota(jnp.int32, (_SUB, _TAIL), 0)
    l_t = lax.broadcasted_iota(jnp.int32, (_SUB, _TAIL), 1)
    cnt_t = (row0 + s_t) * np.int32(_V) + l_t + np.int32(_NCH * _W)
    d_t = _race_d(xs_t, cnt_t, ntinv2)
    m2 = jnp.max(d_t, axis=1, keepdims=True)
    sel2 = jnp.min(
        jnp.where(d_t == m2, cnt_t, _BIG_IDX), axis=1, keepdims=True
    )

    sel = jnp.where(
        m2 > m1, sel2, jnp.where(m1 > m2, sel1, jnp.minimum(sel1, sel2))
    )
    col = sel - (row0 + lax.broadcasted_iota(jnp.int32, (_SUB, 1), 0)) * np.int32(_V)
    o_ref[...] = jnp.broadcast_to(col, (_SUB, 128))


def kernel(logits, temperatures):
    tinv = (np.float32(-np.log2(np.e)) / temperatures).reshape(_B, 1)
    out = pl.pallas_call(
        _rowgroup_body,
        grid=(_B // _SUB,),
        in_specs=[
            pl.BlockSpec((_SUB, _V), lambda g: (g, 0)),
            pl.BlockSpec((_SUB, 1), lambda g: (g, 0)),
        ],
        out_specs=pl.BlockSpec((_SUB, 128), lambda g: (g, 0)),
        out_shape=jax.ShapeDtypeStruct((_B, 128), jnp.int32),
    )(logits, tinv)
    return out[:, 0]
